# baseline (speedup 1.0000x reference)
import jax
import jax.numpy as jnp
from jax.experimental import pallas as pl
from jax.experimental.pallas import tpu as pltpu

N_ROWS = 1000000
D = 16
B = 16384

ROWS_PER_STEP = 32
GATHER_GRID = B // ROWS_PER_STEP

_WT_COLS = 128
_WT_ROWS = (N_ROWS * D) // _WT_COLS
_WT_BLK = 5000
_WT_GRID = _WT_ROWS // _WT_BLK


def _gather_body(idx_ref, w_hbm, inp_ref, tar_ref, est_ref, pred_ref,
                 rows_vmem, sems):
    i = pl.program_id(0)
    copies = []
    for k in range(ROWS_PER_STEP):
        idx = idx_ref[i * ROWS_PER_STEP + k]
        c = pltpu.make_async_copy(
            w_hbm.at[pl.ds(idx, 1), :],
            rows_vmem.at[pl.ds(k, 1), :],
            sems.at[k],
        )
        c.start()
        copies.append(c)
    for c in copies:
        c.wait()

    rows = rows_vmem[...]
    z = jnp.sum(inp_ref[...] * rows, axis=1, keepdims=True)
    est = jax.nn.sigmoid(z)
    est_ref[...] = est
    s = -jnp.sum(tar_ref[...] * jnp.log(est))

    @pl.when(i == 0)
    def _():
        pred_ref[0, 0] = s

    @pl.when(i != 0)
    def _():
        pred_ref[0, 0] += s


def _wt_body(w_ref, out_ref):
    i = pl.program_id(0)
    w = w_ref[...]
    d = w - jnp.clip(w, 0.0, 1.0)
    s = jnp.sum(d * d)

    @pl.when(i == 0)
    def _():
        out_ref[0, 0] = s

    @pl.when(i != 0)
    def _():
        out_ref[0, 0] += s


def kernel(inp, tar, rules_lst, W):
    grid_spec = pltpu.PrefetchScalarGridSpec(
        num_scalar_prefetch=1,
        grid=(GATHER_GRID,),
        in_specs=[
            pl.BlockSpec(memory_space=pltpu.MemorySpace.HBM),
            pl.BlockSpec((ROWS_PER_STEP, D), lambda i, idx_ref: (i, 0)),
            pl.BlockSpec((ROWS_PER_STEP, 1), lambda i, idx_ref: (i, 0)),
        ],
        out_specs=[
            pl.BlockSpec((ROWS_PER_STEP, 1), lambda i, idx_ref: (i, 0)),
            pl.BlockSpec(memory_space=pltpu.MemorySpace.SMEM),
        ],
        scratch_shapes=[
            pltpu.VMEM((ROWS_PER_STEP, D), jnp.float32),
            pltpu.SemaphoreType.DMA((ROWS_PER_STEP,)),
        ],
    )
    est2d, pred = pl.pallas_call(
        _gather_body,
        grid_spec=grid_spec,
        out_shape=[
            jax.ShapeDtypeStruct((B, 1), jnp.float32),
            jax.ShapeDtypeStruct((1, 1), jnp.float32),
        ],
    )(rules_lst, W, inp, tar.reshape(B, 1))

    wt = pl.pallas_call(
        _wt_body,
        grid=(_WT_GRID,),
        in_specs=[pl.BlockSpec((_WT_BLK, _WT_COLS), lambda i: (i, 0))],
        out_specs=pl.BlockSpec(memory_space=pltpu.MemorySpace.SMEM),
        out_shape=jax.ShapeDtypeStruct((1, 1), jnp.float32),
    )(W.reshape(_WT_ROWS, _WT_COLS))

    return est2d.reshape(B), pred[0, 0], wt[0, 0]


# speedup vs baseline: 1.4729x; 1.4729x over previous
import jax
import jax.numpy as jnp
from jax.experimental import pallas as pl
from jax.experimental.pallas import tpu as pltpu

N_ROWS = 1000000
D = 16
B = 16384

ROWS_PER_STEP = 128
GATHER_GRID = B // ROWS_PER_STEP

_WT_COLS = 128
_WT_ROWS = (N_ROWS * D) // _WT_COLS
_WT_BLK = 5000
_WT_GRID = _WT_ROWS // _WT_BLK


def _gather_body(idx_ref, w_hbm, inp_ref, tar_ref, est_ref, pred_ref,
                 rows_vmem, sems):
    i = pl.program_id(0)
    copies = []
    for k in range(ROWS_PER_STEP):
        idx = idx_ref[i * ROWS_PER_STEP + k]
        c = pltpu.make_async_copy(
            w_hbm.at[pl.ds(idx, 1), :],
            rows_vmem.at[pl.ds(k, 1), :],
            sems.at[k],
        )
        c.start()
        copies.append(c)
    for c in copies:
        c.wait()

    rows = rows_vmem[...]
    z = jnp.sum(inp_ref[...] * rows, axis=1, keepdims=True)
    est = jax.nn.sigmoid(z)
    est_ref[...] = est
    s = -jnp.sum(tar_ref[...] * jnp.log(est))

    @pl.when(i == 0)
    def _():
        pred_ref[0, 0] = s

    @pl.when(i != 0)
    def _():
        pred_ref[0, 0] += s


def _wt_body(w_ref, out_ref):
    i = pl.program_id(0)
    w = w_ref[...]
    d = w - jnp.clip(w, 0.0, 1.0)
    s = jnp.sum(d * d)

    @pl.when(i == 0)
    def _():
        out_ref[0, 0] = s

    @pl.when(i != 0)
    def _():
        out_ref[0, 0] += s


def kernel(inp, tar, rules_lst, W):
    grid_spec = pltpu.PrefetchScalarGridSpec(
        num_scalar_prefetch=1,
        grid=(GATHER_GRID,),
        in_specs=[
            pl.BlockSpec(memory_space=pltpu.MemorySpace.HBM),
            pl.BlockSpec((ROWS_PER_STEP, D), lambda i, idx_ref: (i, 0)),
            pl.BlockSpec((ROWS_PER_STEP, 1), lambda i, idx_ref: (i, 0)),
        ],
        out_specs=[
            pl.BlockSpec((ROWS_PER_STEP, 1), lambda i, idx_ref: (i, 0)),
            pl.BlockSpec(memory_space=pltpu.MemorySpace.SMEM),
        ],
        scratch_shapes=[
            pltpu.VMEM((ROWS_PER_STEP, D), jnp.float32),
            pltpu.SemaphoreType.DMA((ROWS_PER_STEP,)),
        ],
    )
    est2d, pred = pl.pallas_call(
        _gather_body,
        grid_spec=grid_spec,
        out_shape=[
            jax.ShapeDtypeStruct((B, 1), jnp.float32),
            jax.ShapeDtypeStruct((1, 1), jnp.float32),
        ],
    )(rules_lst, W, inp, tar.reshape(B, 1))

    wt = pl.pallas_call(
        _wt_body,
        grid=(_WT_GRID,),
        in_specs=[pl.BlockSpec((_WT_BLK, _WT_COLS), lambda i: (i, 0))],
        out_specs=pl.BlockSpec(memory_space=pltpu.MemorySpace.SMEM),
        out_shape=jax.ShapeDtypeStruct((1, 1), jnp.float32),
    )(W.reshape(_WT_ROWS, _WT_COLS))

    return est2d.reshape(B), pred[0, 0], wt[0, 0]


# speedup vs baseline: 1.6007x; 1.0868x over previous
import jax
import jax.numpy as jnp
from jax.experimental import pallas as pl
from jax.experimental.pallas import tpu as pltpu

N_ROWS = 1000000
D = 16
B = 16384

ROWS_PER_STEP = 256
GATHER_GRID = B // ROWS_PER_STEP

_WT_COLS = 128
_WT_ROWS = (N_ROWS * D) // _WT_COLS
_WT_BLK = 5000
_WT_GRID = _WT_ROWS // _WT_BLK


def _gather_body(idx_ref, w_hbm, inp_ref, tar_ref, est_ref, pred_ref,
                 rows_vmem, sems):
    i = pl.program_id(0)
    copies = []
    for k in range(ROWS_PER_STEP):
        idx = idx_ref[i * ROWS_PER_STEP + k]
        c = pltpu.make_async_copy(
            w_hbm.at[pl.ds(idx, 1), :],
            rows_vmem.at[pl.ds(k, 1), :],
            sems.at[k],
        )
        c.start()
        copies.append(c)
    for c in copies:
        c.wait()

    rows = rows_vmem[...]
    z = jnp.sum(inp_ref[...] * rows, axis=1, keepdims=True)
    est = jax.nn.sigmoid(z)
    est_ref[...] = est
    s = -jnp.sum(tar_ref[...] * jnp.log(est))

    @pl.when(i == 0)
    def _():
        pred_ref[0, 0] = s

    @pl.when(i != 0)
    def _():
        pred_ref[0, 0] += s


def _wt_body(w_ref, out_ref):
    i = pl.program_id(0)
    w = w_ref[...]
    d = w - jnp.clip(w, 0.0, 1.0)
    s = jnp.sum(d * d)

    @pl.when(i == 0)
    def _():
        out_ref[0, 0] = s

    @pl.when(i != 0)
    def _():
        out_ref[0, 0] += s


def kernel(inp, tar, rules_lst, W):
    grid_spec = pltpu.PrefetchScalarGridSpec(
        num_scalar_prefetch=1,
        grid=(GATHER_GRID,),
        in_specs=[
            pl.BlockSpec(memory_space=pltpu.MemorySpace.HBM),
            pl.BlockSpec((ROWS_PER_STEP, D), lambda i, idx_ref: (i, 0)),
            pl.BlockSpec((ROWS_PER_STEP, 1), lambda i, idx_ref: (i, 0)),
        ],
        out_specs=[
            pl.BlockSpec((ROWS_PER_STEP, 1), lambda i, idx_ref: (i, 0)),
            pl.BlockSpec(memory_space=pltpu.MemorySpace.SMEM),
        ],
        scratch_shapes=[
            pltpu.VMEM((ROWS_PER_STEP, D), jnp.float32),
            pltpu.SemaphoreType.DMA((ROWS_PER_STEP,)),
        ],
    )
    est2d, pred = pl.pallas_call(
        _gather_body,
        grid_spec=grid_spec,
        out_shape=[
            jax.ShapeDtypeStruct((B, 1), jnp.float32),
            jax.ShapeDtypeStruct((1, 1), jnp.float32),
        ],
    )(rules_lst, W, inp, tar.reshape(B, 1))

    wt = pl.pallas_call(
        _wt_body,
        grid=(_WT_GRID,),
        in_specs=[pl.BlockSpec((_WT_BLK, _WT_COLS), lambda i: (i, 0))],
        out_specs=pl.BlockSpec(memory_space=pltpu.MemorySpace.SMEM),
        out_shape=jax.ShapeDtypeStruct((1, 1), jnp.float32),
    )(W.reshape(_WT_ROWS, _WT_COLS))

    return est2d.reshape(B), pred[0, 0], wt[0, 0]
